# Initial kernel scaffold; baseline (speedup 1.0000x reference)
import jax
import jax.numpy as jnp
from jax.experimental import pallas as pl


def kernel(params, input_ids):
    raise NotImplementedError("write your pallas kernel here")


# speedup vs baseline: 1.0579x; 1.0579x over previous
import functools

import jax
import jax.numpy as jnp
from jax.experimental import pallas as pl
from jax.experimental.pallas import tpu as pltpu
from jax.experimental.pallas import tpu_sc as plsc

D = 768
NH = 12
DH = D // NH
E = 8
TK = 2
FF = 4 * D
S = 2048
V = 32000

BM = 256
NB = (TK * S) // BM + E
P = NB * BM
FFB = 1536
NFF = FF // FFB

_NC = 2
_NS = 16
_NW = _NC * _NS


def _sc_mesh():
    return plsc.VectorSubcoreMesh(core_axis_name="c", subcore_axis_name="s")


def _sc_gather_rows(table, idx):
    B = idx.shape[0]
    d = table.shape[1]
    bw = B // _NW

    @functools.partial(
        pl.kernel,
        mesh=_sc_mesh(),
        out_type=jax.ShapeDtypeStruct((B, d), table.dtype),
        scratch_types=[
            pltpu.VMEM((bw,), jnp.int32),
            pltpu.VMEM((bw, d), table.dtype),
            pltpu.SemaphoreType.DMA,
        ],
    )
    def k(table_hbm, idx_hbm, out_hbm, idx_v, rows_v, sem):
        wid = jax.lax.axis_index("s") * _NC + jax.lax.axis_index("c")
        base = wid * bw
        pltpu.sync_copy(idx_hbm.at[pl.ds(base, bw)], idx_v)
        pltpu.async_copy(table_hbm.at[idx_v], rows_v, sem).wait()
        pltpu.sync_copy(rows_v, out_hbm.at[pl.ds(base, bw)])

    return k(table, idx)


def _sc_scatter_rows(rows, idx2d, n_out):
    nrows, d = rows.shape
    nw, bw = idx2d.shape
    nsrc = nrows // bw

    @functools.partial(
        pl.kernel,
        mesh=_sc_mesh(),
        out_type=jax.ShapeDtypeStruct((n_out, d), rows.dtype),
        scratch_types=[
            pltpu.VMEM((bw,), jnp.int32),
            pltpu.VMEM((bw, d), rows.dtype),
            pltpu.SemaphoreType.DMA,
        ],
    )
    def k(x_hbm, i_hbm, o_hbm, idx_v, rows_v, sem):
        wid = jax.lax.axis_index("s") * _NC + jax.lax.axis_index("c")
        src = jax.lax.rem(wid, nsrc) * bw
        pltpu.sync_copy(i_hbm.at[wid], idx_v)
        pltpu.sync_copy(x_hbm.at[pl.ds(src, bw)], rows_v)
        pltpu.async_copy(rows_v, o_hbm.at[idx_v], sem).wait()

    return k(rows, idx2d)


def _add(a, b):
    def body(a_ref, b_ref, o_ref):
        o_ref[...] = a_ref[...] + b_ref[...]

    return pl.pallas_call(
        body, out_shape=jax.ShapeDtypeStruct(a.shape, a.dtype)
    )(a, b)


def _ln_body(x, g, b):
    m = jnp.mean(x, axis=-1, keepdims=True)
    v = jnp.mean((x - m) * (x - m), axis=-1, keepdims=True)
    return (x - m) * jax.lax.rsqrt(v + 1e-5) * g + b


DHP = 128
QKVP = 3 * NH * DHP
CTXP = NH * DHP


def _ln_qkv(x, g, b, wqkv_pad, bqkv_pad):
    CB = QKVP // 4

    def body(x_ref, g_ref, b_ref, w_ref, bias_ref, o_ref):
        h = _ln_body(x_ref[...], g_ref[...], b_ref[...])
        o_ref[...] = (
            jax.lax.dot_general(
                h, w_ref[...], (((1,), (1,)), ((), ())),
                preferred_element_type=jnp.float32,
            )
            + bias_ref[...]
        )

    return pl.pallas_call(
        body,
        grid=(QKVP // CB,),
        in_specs=[
            pl.BlockSpec((S, D), lambda c: (0, 0)),
            pl.BlockSpec((1, D), lambda c: (0, 0)),
            pl.BlockSpec((1, D), lambda c: (0, 0)),
            pl.BlockSpec((CB, D), lambda c: (c, 0)),
            pl.BlockSpec((1, CB), lambda c: (0, c)),
        ],
        out_specs=pl.BlockSpec((S, CB), lambda c: (0, c)),
        out_shape=jax.ShapeDtypeStruct((S, QKVP), jnp.float32),
    )(x, g, b, wqkv_pad, bqkv_pad)


def _attention(qkv):
    BQ = 512
    scale = 0.125

    def body(q_ref, k_ref, v_ref, o_ref):
        s = jax.lax.dot_general(
            q_ref[...], k_ref[...], (((1,), (1,)), ((), ())),
            preferred_element_type=jnp.float32,
        ) * scale
        s = s - jnp.max(s, axis=-1, keepdims=True)
        e = jnp.exp(s)
        p = e / jnp.sum(e, axis=-1, keepdims=True)
        o_ref[...] = jnp.dot(p, v_ref[...], preferred_element_type=jnp.float32)

    return pl.pallas_call(
        body,
        grid=(NH, S // BQ),
        in_specs=[
            pl.BlockSpec((BQ, DHP), lambda h, i: (i, h)),
            pl.BlockSpec((S, DHP), lambda h, i: (0, NH + h)),
            pl.BlockSpec((S, DHP), lambda h, i: (0, 2 * NH + h)),
        ],
        out_specs=pl.BlockSpec((BQ, DHP), lambda h, i: (i, h)),
        out_shape=jax.ShapeDtypeStruct((S, CTXP), jnp.float32),
    )(qkv, qkv, qkv)


def _proj_res(ctx, wo_pad, bo, res):
    def body(c_ref, w_ref, b_ref, r_ref, o_ref):
        o_ref[...] = (
            jax.lax.dot_general(
                c_ref[...], w_ref[...], (((1,), (1,)), ((), ())),
                preferred_element_type=jnp.float32,
            )
            + b_ref[...]
            + r_ref[...]
        )

    return pl.pallas_call(
        body, out_shape=jax.ShapeDtypeStruct((S, D), jnp.float32)
    )(ctx, wo_pad, bo, res)


def _ln_router(x, g, b, wr, br):
    def body(x_ref, g_ref, b_ref, wr_ref, br_ref,
             h_ref, pos_ref, gate_ref, be_ref):
        h = _ln_body(x_ref[...], g_ref[...], b_ref[...])
        h_ref[...] = h
        logits = (
            jax.lax.dot_general(
                h, wr_ref[...], (((1,), (1,)), ((), ())),
                preferred_element_type=jnp.float32,
            )
            + br_ref[...]
        )
        mx = jnp.max(logits, axis=-1, keepdims=True)
        ex = jnp.exp(logits - mx)
        probs = ex / jnp.sum(ex, axis=-1, keepdims=True)

        ecols = jax.lax.broadcasted_iota(jnp.int32, (S, E), 1)
        m1 = jnp.max(probs, axis=-1, keepdims=True)
        i1 = jnp.min(jnp.where(probs == m1, ecols, E), axis=-1, keepdims=True)
        probs2 = jnp.where(ecols == i1, -1.0, probs)
        m2 = jnp.max(probs2, axis=-1, keepdims=True)
        i2 = jnp.min(jnp.where(probs2 == m2, ecols, E), axis=-1, keepdims=True)
        ssum = m1 + m2
        gate_ref[...] = jnp.concatenate([m1 / ssum, m2 / ssum], axis=1)

        sel1 = (ecols == i1).astype(jnp.float32)
        sel2 = (ecols == i2).astype(jnp.float32)
        cnt = sel1 + sel2

        rr = jax.lax.broadcasted_iota(jnp.int32, (S, S), 0)
        cc = jax.lax.broadcasted_iota(jnp.int32, (S, S), 1)
        lt = (cc < rr).astype(jnp.float32)
        cnt_before = jnp.dot(lt, cnt, preferred_element_type=jnp.float32)

        counts = jnp.sum(cnt, axis=0, keepdims=True)
        pc = jnp.ceil(counts * (1.0 / BM)) * BM
        er = jax.lax.broadcasted_iota(jnp.int32, (E, E), 0)
        ec = jax.lax.broadcasted_iota(jnp.int32, (E, E), 1)
        off = jnp.dot(pc, (er < ec).astype(jnp.float32),
                      preferred_element_type=jnp.float32)
        cum = jnp.dot(pc, (er <= ec).astype(jnp.float32),
                      preferred_element_type=jnp.float32)

        base = off + cnt_before
        pos1 = jnp.sum(jnp.where(ecols == i1, base, 0.0), axis=-1, keepdims=True)
        pos2 = jnp.sum(jnp.where(ecols == i2, base, 0.0), axis=-1, keepdims=True)
        pos_ref[...] = jnp.concatenate([pos1, pos2], axis=1).astype(jnp.int32)

        blk = jax.lax.broadcasted_iota(jnp.int32, (NB, E), 0) * BM
        be = jnp.sum((blk.astype(jnp.float32) >= cum).astype(jnp.float32),
                     axis=-1, keepdims=True)
        be_ref[...] = jnp.minimum(be, E - 1).astype(jnp.int32)

    return pl.pallas_call(
        body,
        out_shape=(
            jax.ShapeDtypeStruct((S, D), jnp.float32),
            jax.ShapeDtypeStruct((S, 2), jnp.int32),
            jax.ShapeDtypeStruct((S, 2), jnp.float32),
            jax.ShapeDtypeStruct((NB, 1), jnp.int32),
        ),
    )(x, g, b, wr, br)

def _grouped_ffn(xb, w1, b1, w2, b2, be):
    def body(be_ref, x_ref, w1_ref, b1_ref, w2_ref, b2_ref, o_ref):
        j = pl.program_id(1)
        a = (
            jnp.dot(x_ref[...], w1_ref[0], preferred_element_type=jnp.float32)
            + b1_ref[0]
        )
        a = jax.nn.gelu(a)
        h2 = jnp.dot(a, w2_ref[0], preferred_element_type=jnp.float32)

        @pl.when(j == 0)
        def _():
            o_ref[...] = h2 + b2_ref[0]

        @pl.when(j != 0)
        def _():
            o_ref[...] += h2

    grid_spec = pltpu.PrefetchScalarGridSpec(
        num_scalar_prefetch=1,
        grid=(NB, NFF),
        in_specs=[
            pl.BlockSpec((BM, D), lambda i, j, be: (i, 0)),
            pl.BlockSpec((1, D, FFB), lambda i, j, be: (be[i], 0, j)),
            pl.BlockSpec((1, 1, FFB), lambda i, j, be: (be[i], 0, j)),
            pl.BlockSpec((1, FFB, D), lambda i, j, be: (be[i], j, 0)),
            pl.BlockSpec((1, 1, D), lambda i, j, be: (be[i], 0, 0)),
        ],
        out_specs=pl.BlockSpec((BM, D), lambda i, j, be: (i, 0)),
    )
    return pl.pallas_call(
        body,
        grid_spec=grid_spec,
        out_shape=jax.ShapeDtypeStruct((P, D), jnp.float32),
    )(be, xb, w1, b1.reshape(E, 1, FF), w2, b2.reshape(E, 1, D))


def _combine(res, gath, gates):
    def body(r_ref, a_ref, b_ref, w_ref, o_ref):
        o_ref[...] = (
            r_ref[...]
            + w_ref[:, 0:1] * a_ref[...]
            + w_ref[:, 1:2] * b_ref[...]
        )

    return pl.pallas_call(
        body,
        grid=(1,),
        in_specs=[
            pl.BlockSpec((S, D), lambda i: (0, 0)),
            pl.BlockSpec((S, D), lambda i: (0, 0)),
            pl.BlockSpec((S, D), lambda i: (1, 0)),
            pl.BlockSpec((S, TK), lambda i: (0, 0)),
        ],
        out_specs=pl.BlockSpec((S, D), lambda i: (0, 0)),
        out_shape=jax.ShapeDtypeStruct((S, D), jnp.float32),
    )(res, gath, gath, gates)


def _lm_head(x, g, b, wlm, blm):
    BR = 1024
    CB = 1280

    def body(x_ref, g_ref, b_ref, w_ref, bias_ref, o_ref):
        xx = x_ref[...]
        m = jnp.mean(xx, axis=-1, keepdims=True)
        v = jnp.mean((xx - m) * (xx - m), axis=-1, keepdims=True)
        h = (xx - m) * jax.lax.rsqrt(v + 1e-5) * g_ref[...] + b_ref[...]
        o_ref[...] = (
            jax.lax.dot_general(
                h, w_ref[...], (((1,), (1,)), ((), ())),
                preferred_element_type=jnp.float32,
            )
            + bias_ref[...]
        )

    return pl.pallas_call(
        body,
        grid=(V // CB, S // BR),
        in_specs=[
            pl.BlockSpec((BR, D), lambda c, r: (r, 0)),
            pl.BlockSpec((1, D), lambda c, r: (0, 0)),
            pl.BlockSpec((1, D), lambda c, r: (0, 0)),
            pl.BlockSpec((CB, D), lambda c, r: (c, 0)),
            pl.BlockSpec((1, CB), lambda c, r: (0, c)),
        ],
        out_specs=pl.BlockSpec((BR, CB), lambda c, r: (r, c)),
        out_shape=jax.ShapeDtypeStruct((S, V), jnp.float32),
    )(x, g, b, wlm, blm)


def kernel(params, input_ids):
    p = params
    ids = input_ids.reshape(S).astype(jnp.int32)

    emb = _sc_gather_rows(p['tok_emb'], ids)
    x = _add(emb, p['pos_emb'])

    for blk in p['blocks']:
        w_re = blk['Wqkv'].reshape(3 * NH, DH, D)
        wqkv_pad = jnp.concatenate(
            [w_re, jnp.zeros((3 * NH, DHP - DH, D), jnp.float32)], axis=1
        ).reshape(QKVP, D)
        b_re = blk['bqkv'].reshape(3 * NH, DH)
        bqkv_pad = jnp.concatenate(
            [b_re, jnp.zeros((3 * NH, DHP - DH), jnp.float32)], axis=1
        ).reshape(1, QKVP)
        wo_re = blk['Wo'].reshape(D, NH, DH)
        wo_pad = jnp.concatenate(
            [wo_re, jnp.zeros((D, NH, DHP - DH), jnp.float32)], axis=2
        ).reshape(D, CTXP)

        qkv = _ln_qkv(x, blk['ln1_g'].reshape(1, D), blk['ln1_b'].reshape(1, D),
                      wqkv_pad, bqkv_pad)
        ctx = _attention(qkv)
        x = _proj_res(ctx, wo_pad, blk['bo'].reshape(1, D), x)

        h_ln, pos2, gates, be2 = _ln_router(
            x, blk['ln2_g'].reshape(1, D), blk['ln2_b'].reshape(1, D),
            blk['Wr'], blk['br'].reshape(1, E))
        pos_flat = jnp.transpose(pos2).reshape(TK * S)
        be = be2.reshape(NB)

        disp = _sc_scatter_rows(h_ln, pos_flat.reshape(_NW, (TK * S) // _NW), P)
        eout = _grouped_ffn(disp, blk['W1'], blk['b1'], blk['W2'], blk['b2'], be)
        gath = _sc_gather_rows(eout, pos_flat)
        x = _combine(x, gath, gates)

    out = _lm_head(x, p['lnf_g'].reshape(1, D),
                   p['lnf_b'].reshape(1, D), p['Wlm'], p['blm'].reshape(1, V))
    return out.reshape(1, S, V)
